# baseline (speedup 1.0000x reference)
import functools

import jax
import jax.numpy as jnp
from jax import lax
from jax.experimental import pallas as pl
from jax.experimental.pallas import tpu as pltpu
from jax.experimental.pallas import tpu_sc as plsc

NC = 2
NS = 16
NW = NC * NS
L = 16
D = 128
GPB = 25
GU = 5


@functools.lru_cache(maxsize=None)
def _build_sc(e_pad: int, nsb: int, sym: bool):
    epw = nsb * GPB * L
    mesh = plsc.VectorSubcoreMesh(
        core_axis_name="c", subcore_axis_name="s",
        num_cores=NC, num_subcores=NS)

    def body(src_hbm, typ_hbm, rel_hbm, uns_hbm, att_hbm,
             nid_hbm, ent_hbm, rtab_hbm,
             part_hbm, cnt_hbm,
             dstb, attv, unsv, idxbuf, srcbuf, typbuf, relbuf, nid16,
             ent_rows, rel_rows, acc, cntv,
             sem_s, sem_t, sem_r, sem_n, sem_e, sem_rr):
        wid = lax.axis_index("s") * NC + lax.axis_index("c")
        pltpu.sync_copy(att_hbm, attv)
        pltpu.sync_copy(uns_hbm, unsv)
        if sym:
            half = e_pad // 2
            shift = jnp.where(wid < NW // 2, half, -half)
        else:
            shift = e_pad
        pltpu.sync_copy(src_hbm.at[pl.ds(wid * epw + shift, epw)], dstb)
        zf = jnp.zeros((L,), jnp.float32)
        for j in range(32):
            acc[pl.ds(j * L, L)] = zf
        cntv[:] = zf
        unseen = unsv[:]
        lanes = lax.iota(jnp.int32, L)

        def sb_body(sb, carry0):
            base_g = sb * GPB

            def fast_body(k, m):
                base = (base_g + k * GU) * L
                for u in range(GU):
                    d = dstb[pl.ds(base + u * L, L)]
                    m = m | (d == unseen).astype(jnp.int32)
                return m

            orv = lax.fori_loop(0, GPB // GU, fast_body,
                                jnp.zeros((L,), jnp.int32))

            @pl.when(jnp.max(orv) > 0)
            def _rescan():
                def dg_body(g, c):
                    row = base_g + g
                    d = dstb[pl.ds(row * L, L)]
                    mask = d == unseen

                    @pl.when(jnp.any(mask))
                    def _hit_group():
                        ebase = wid * epw + row * L
                        idxbuf[:] = ebase + lanes
                        cp_s = pltpu.async_copy(src_hbm.at[idxbuf], srcbuf,
                                                sem_s)
                        cp_t = pltpu.async_copy(typ_hbm.at[idxbuf], typbuf,
                                                sem_t)
                        cp_r = pltpu.async_copy(rel_hbm.at[idxbuf], relbuf,
                                                sem_r)
                        cp_s.wait()
                        cp_t.wait()
                        cp_r.wait()
                        cp_n = pltpu.async_copy(nid_hbm.at[srcbuf], nid16,
                                                sem_n)
                        cp_rr = pltpu.async_copy(rtab_hbm.at[relbuf], rel_rows,
                                                 sem_rr)
                        cp_n.wait()
                        cp_e = pltpu.async_copy(ent_hbm.at[nid16], ent_rows,
                                                sem_e)
                        typv = typbuf[:]
                        maskf = jnp.where(mask, 1.0, 0.0)
                        w0 = plsc.load_gather(attv.at[:], [typv * 2]) * maskf
                        w1 = plsc.load_gather(attv.at[:], [typv * 2 + 1]) * maskf
                        cntv[:] = cntv[:] + maskf
                        cp_rr.wait()
                        cp_e.wait()

                        def lane_body(t, c2):
                            tv = jnp.full((L,), t, jnp.int32)
                            w0s = w0[tv]
                            w1s = w1[tv]
                            for j in range(8):
                                sl = pl.ds(j * L, L)
                                e = ent_rows[t, sl]
                                r = rel_rows[t, sl]
                                acc[sl] += w0s * e
                                acc[pl.ds(128 + j * L, L)] += w0s * r
                                acc[pl.ds(256 + j * L, L)] += w1s * e
                                acc[pl.ds(384 + j * L, L)] += w1s * r
                            return c2

                        lax.fori_loop(0, L, lane_body, 0)

                    return c

                lax.fori_loop(0, GPB, dg_body, 0)

            return carry0

        lax.fori_loop(0, nsb, sb_body, 0)
        pltpu.sync_copy(acc, part_hbm.at[pl.ds(wid * 512, 512)])
        pltpu.sync_copy(cntv, cnt_hbm.at[pl.ds(wid * L, L)])

    return pl.kernel(
        body,
        out_type=[jax.ShapeDtypeStruct((NW * 512,), jnp.float32),
                  jax.ShapeDtypeStruct((NW * L,), jnp.float32)],
        mesh=mesh,
        compiler_params=pltpu.CompilerParams(needs_layout_passes=False),
        scratch_types=[
            pltpu.VMEM((epw,), jnp.int32),
            pltpu.VMEM((800,), jnp.float32),
            pltpu.VMEM((L,), jnp.int32),
            pltpu.VMEM((L,), jnp.int32),
            pltpu.VMEM((L,), jnp.int32),
            pltpu.VMEM((L,), jnp.int32),
            pltpu.VMEM((L,), jnp.int32),
            pltpu.VMEM((L,), jnp.int32),
            pltpu.VMEM((L, D), jnp.float32),
            pltpu.VMEM((L, D), jnp.float32),
            pltpu.VMEM((512,), jnp.float32),
            pltpu.VMEM((L,), jnp.float32),
            pltpu.SemaphoreType.DMA,
            pltpu.SemaphoreType.DMA,
            pltpu.SemaphoreType.DMA,
            pltpu.SemaphoreType.DMA,
            pltpu.SemaphoreType.DMA,
            pltpu.SemaphoreType.DMA,
        ],
    )


def _combine_body(p_ref, c_ref, b_ref, o_ref):
    p = p_ref[...]
    v4 = jnp.sum(p.reshape(NW, 4, D), axis=0)
    cnt = jnp.sum(c_ref[...])
    v0 = jnp.concatenate([v4[0:1], v4[1:2]], axis=1)
    v1 = jnp.concatenate([v4[2:3], v4[3:4]], axis=1)
    b = b_ref[...]
    r = jnp.dot(v0, b[0], preferred_element_type=jnp.float32)
    r = r + jnp.dot(v1, b[1], preferred_element_type=jnp.float32)
    o_ref[...] = r / jnp.maximum(cnt, 1.0)


_combine = pl.pallas_call(
    _combine_body,
    out_shape=jax.ShapeDtypeStruct((1, D), jnp.float32),
)


def kernel(node_id, edge_index, edge_type, rel_index, unseen_index,
           entity_embedding, relation_embedding, att, basis):
    e = edge_type.shape[0]
    spb = GPB * L
    nsb = -(-e // (NW * spb))
    e_pad = NW * nsb * spb
    pe = e_pad - e
    if pe == 0 and e % 2 == 0:
        src = edge_index[0]
        typ = edge_type
        rel = rel_index
        sym = True
    else:
        srcp = jnp.pad(edge_index[0], (0, pe))
        dstp = jnp.pad(edge_index[1], (0, pe), constant_values=-1)
        src = jnp.concatenate([srcp, dstp])
        typ = jnp.pad(edge_type, (0, pe))
        rel = jnp.pad(rel_index, (0, pe))
        sym = False
    uns = jnp.full((L,), unseen_index, dtype=jnp.int32)
    att_flat = att.reshape(-1)
    part, cnt = _build_sc(e_pad, nsb, sym)(
        src, typ, rel, uns, att_flat,
        node_id, entity_embedding, relation_embedding)
    out = _combine(part.reshape(D, D), cnt.reshape(4, D), basis)
    return out[0]


# speedup vs baseline: 1.2969x; 1.2969x over previous
import functools

import jax
import jax.numpy as jnp
from jax import lax
from jax.experimental import pallas as pl
from jax.experimental.pallas import tpu as pltpu
from jax.experimental.pallas import tpu_sc as plsc

NC = 2
NS = 16
NW = NC * NS
L = 16
D = 128
GPB = 25
GU = 5


@functools.lru_cache(maxsize=None)
def _build_sc(e_pad: int, nsb: int, sym: bool):
    epw = nsb * GPB * L
    mesh = plsc.VectorSubcoreMesh(
        core_axis_name="c", subcore_axis_name="s",
        num_cores=NC, num_subcores=NS)

    def body(src_hbm, typ_hbm, rel_hbm, uns_hbm, att_hbm,
             nid_hbm, ent_hbm, rtab_hbm,
             part_hbm, cnt_hbm,
             dstb, attv, unsv, idxbuf, srcbuf, typbuf, relbuf, nid16,
             ent_rows, rel_rows, acc, cntv,
             sem_s, sem_t, sem_r, sem_n, sem_e, sem_rr):
        wid = lax.axis_index("s") * NC + lax.axis_index("c")
        pltpu.sync_copy(att_hbm, attv)
        pltpu.sync_copy(uns_hbm, unsv)
        if sym:
            half = e_pad // 2
            shift = jnp.where(wid < NW // 2, half, -half)
        else:
            shift = e_pad
        pltpu.sync_copy(src_hbm.at[pl.ds(wid * epw + shift, epw)], dstb)
        zf = jnp.zeros((L,), jnp.float32)
        for j in range(32):
            acc[pl.ds(j * L, L)] = zf
        cntv[:] = zf
        unseen = unsv[:]
        lanes = lax.iota(jnp.int32, L)

        def sb_body(sb, carry0):
            base_g = sb * GPB

            def fast_body(k, m):
                base = (base_g + k * GU) * L
                for u in range(GU):
                    d = dstb[pl.ds(base + u * L, L)]
                    m = m | (d == unseen).astype(jnp.int32)
                return m

            orv = lax.fori_loop(0, GPB // GU, fast_body,
                                jnp.zeros((L,), jnp.int32))

            @pl.when(jnp.max(orv) > 0)
            def _rescan():
                def dg_body(g, c):
                    row = base_g + g
                    d = dstb[pl.ds(row * L, L)]
                    mask = d == unseen

                    @pl.when(jnp.any(mask))
                    def _hit_group():
                        ebase = wid * epw + row * L
                        idxbuf[:] = ebase + lanes
                        cp_s = pltpu.async_copy(src_hbm.at[idxbuf], srcbuf,
                                                sem_s)
                        cp_t = pltpu.async_copy(typ_hbm.at[idxbuf], typbuf,
                                                sem_t)
                        cp_r = pltpu.async_copy(rel_hbm.at[idxbuf], relbuf,
                                                sem_r)
                        cp_s.wait()
                        cp_t.wait()
                        cp_r.wait()
                        cp_n = pltpu.async_copy(nid_hbm.at[srcbuf], nid16,
                                                sem_n)
                        cp_rr = pltpu.async_copy(rtab_hbm.at[relbuf], rel_rows,
                                                 sem_rr)
                        cp_n.wait()
                        cp_e = pltpu.async_copy(ent_hbm.at[nid16], ent_rows,
                                                sem_e)
                        typv = typbuf[:]
                        maskf = jnp.where(mask, 1.0, 0.0)
                        w0 = plsc.load_gather(attv.at[:], [typv * 2]) * maskf
                        w1 = plsc.load_gather(attv.at[:], [typv * 2 + 1]) * maskf
                        cntv[:] = cntv[:] + maskf
                        cp_rr.wait()
                        cp_e.wait()

                        def lane_body(t, c2):
                            tv = jnp.full((L,), t, jnp.int32)
                            w0s = w0[tv]
                            w1s = w1[tv]
                            for j in range(8):
                                sl = pl.ds(j * L, L)
                                e = ent_rows[t, sl]
                                r = rel_rows[t, sl]
                                acc[sl] += w0s * e
                                acc[pl.ds(128 + j * L, L)] += w0s * r
                                acc[pl.ds(256 + j * L, L)] += w1s * e
                                acc[pl.ds(384 + j * L, L)] += w1s * r
                            return c2

                        lax.fori_loop(0, L, lane_body, 0)

                    return c

                lax.fori_loop(0, GPB, dg_body, 0)

            return carry0

        lax.fori_loop(0, nsb, sb_body, 0)
        pltpu.sync_copy(acc, part_hbm.at[pl.ds(wid * 512, 512)])
        pltpu.sync_copy(cntv, cnt_hbm.at[pl.ds(wid * L, L)])

    return pl.kernel(
        body,
        out_type=[jax.ShapeDtypeStruct((NW * 512,), jnp.float32),
                  jax.ShapeDtypeStruct((NW * L,), jnp.float32)],
        mesh=mesh,
        compiler_params=pltpu.CompilerParams(needs_layout_passes=False),
        scratch_types=[
            pltpu.VMEM((epw,), jnp.int32),
            pltpu.VMEM((800,), jnp.float32),
            pltpu.VMEM((L,), jnp.int32),
            pltpu.VMEM((L,), jnp.int32),
            pltpu.VMEM((L,), jnp.int32),
            pltpu.VMEM((L,), jnp.int32),
            pltpu.VMEM((L,), jnp.int32),
            pltpu.VMEM((L,), jnp.int32),
            pltpu.VMEM((L, D), jnp.float32),
            pltpu.VMEM((L, D), jnp.float32),
            pltpu.VMEM((512,), jnp.float32),
            pltpu.VMEM((L,), jnp.float32),
            pltpu.SemaphoreType.DMA,
            pltpu.SemaphoreType.DMA,
            pltpu.SemaphoreType.DMA,
            pltpu.SemaphoreType.DMA,
            pltpu.SemaphoreType.DMA,
            pltpu.SemaphoreType.DMA,
        ],
    )


def _combine_body(p_ref, c_ref, b_ref, o_ref):
    p = p_ref[...]
    v4 = jnp.sum(p.reshape(NW, 4, D), axis=0)
    cnt = jnp.sum(c_ref[...])
    v0 = jnp.concatenate([v4[0:1], v4[1:2]], axis=1)
    v1 = jnp.concatenate([v4[2:3], v4[3:4]], axis=1)
    b = b_ref[...]
    r = jnp.dot(v0, b[0], preferred_element_type=jnp.float32)
    r = r + jnp.dot(v1, b[1], preferred_element_type=jnp.float32)
    o_ref[...] = r / jnp.maximum(cnt, 1.0)


_combine = pl.pallas_call(
    _combine_body,
    out_shape=jax.ShapeDtypeStruct((1, D), jnp.float32),
)


def kernel(node_id, edge_index, edge_type, rel_index, unseen_index,
           entity_embedding, relation_embedding, att, basis):
    e = edge_type.shape[0]
    spb = GPB * L
    nsb = -(-e // (NW * spb))
    e_pad = NW * nsb * spb
    pe = e_pad - e
    if pe == 0:
        src = edge_index.reshape(-1)
        typ = edge_type
        rel = rel_index
        sym = False
    else:
        srcp = jnp.pad(edge_index[0], (0, pe))
        dstp = jnp.pad(edge_index[1], (0, pe), constant_values=-1)
        src = jnp.concatenate([srcp, dstp])
        typ = jnp.pad(edge_type, (0, pe))
        rel = jnp.pad(rel_index, (0, pe))
        sym = False
    uns = jnp.full((L,), unseen_index, dtype=jnp.int32)
    att_flat = att.reshape(-1)
    part, cnt = _build_sc(e_pad, nsb, sym)(
        src, typ, rel, uns, att_flat,
        node_id, entity_embedding, relation_embedding)
    out = _combine(part.reshape(D, D), cnt.reshape(4, D), basis)
    return out[0]


# speedup vs baseline: 1.4003x; 1.0798x over previous
import functools

import jax
import jax.numpy as jnp
from jax import lax
from jax.experimental import pallas as pl
from jax.experimental.pallas import tpu as pltpu
from jax.experimental.pallas import tpu_sc as plsc

NC = 2
NS = 16
NW = NC * NS
L = 16
D = 128
GPB = 25
GU = 5


GPB2 = 40
GU2 = 8


@functools.lru_cache(maxsize=None)
def _build_sc_raw(E: int, nsb: int):
    epw = nsb * GPB2 * L
    T = E // 2
    mesh = plsc.VectorSubcoreMesh(
        core_axis_name="c", subcore_axis_name="s",
        num_cores=NC, num_subcores=NS)

    def body(ei_hbm, typ_hbm, rel_hbm, uns_hbm, att_hbm,
             nid_hbm, ent_hbm, rtab_hbm,
             part_hbm, cnt_hbm,
             dstb, attv, unsv, idxbuf, srcwin, srcbuf, typbuf, relbuf, nid16,
             ent_rows, rel_rows, acc, cntv,
             sem_s, sem_t, sem_r, sem_n, sem_e, sem_rr):
        wid = lax.axis_index("s") * NC + lax.axis_index("c")
        pltpu.sync_copy(att_hbm, attv)
        pltpu.sync_copy(uns_hbm, unsv)
        wlow = wid * epw
        s_w = jnp.minimum(wlow, E - epw)
        pltpu.sync_copy(ei_hbm.at[:, pl.ds(s_w, epw)], dstb)
        zf = jnp.zeros((L,), jnp.float32)
        for j in range(32):
            acc[pl.ds(j * L, L)] = zf
        cntv[:] = zf
        unseen = unsv[:]
        lanes = lax.iota(jnp.int32, L)

        def sb_body(sb, carry0):
            base_g = sb * GPB2

            def fast_body(k, m):
                base = (base_g + k * GU2) * L
                for u in range(GU2):
                    d = dstb[1, pl.ds(base + u * L, L)]
                    m = m | (d == unseen).astype(jnp.int32)
                return m

            orv = lax.fori_loop(0, GPB2 // GU2, fast_body,
                                jnp.zeros((L,), jnp.int32))

            @pl.when(jnp.max(orv) > 0)
            def _rescan():
                def dg_body(g, c):
                    row = base_g + g
                    ps = s_w + row * L
                    d = dstb[1, pl.ds(row * L, L)]
                    mask = (d == unseen) & (ps + lanes >= wlow)

                    @pl.when(jnp.any(mask))
                    def _hit_group():
                        idxbuf[:] = ps + lanes
                        cp_t = pltpu.async_copy(typ_hbm.at[idxbuf], typbuf,
                                                sem_t)
                        cp_r = pltpu.async_copy(rel_hbm.at[idxbuf], relbuf,
                                                sem_r)
                        srcbuf[:] = dstb[0, pl.ds(row * L, L)]
                        cp_t.wait()
                        cp_r.wait()
                        cp_n = pltpu.async_copy(nid_hbm.at[srcbuf], nid16,
                                                sem_n)
                        cp_rr = pltpu.async_copy(rtab_hbm.at[relbuf], rel_rows,
                                                 sem_rr)
                        cp_n.wait()
                        cp_e = pltpu.async_copy(ent_hbm.at[nid16], ent_rows,
                                                sem_e)
                        typv = typbuf[:]
                        maskf = jnp.where(mask, 1.0, 0.0)
                        w0 = plsc.load_gather(attv.at[:], [typv * 2]) * maskf
                        w1 = plsc.load_gather(attv.at[:], [typv * 2 + 1]) * maskf
                        cntv[:] = cntv[:] + maskf
                        cp_rr.wait()
                        cp_e.wait()

                        def lane_body(t, c2):
                            tv = jnp.full((L,), t, jnp.int32)
                            w0s = w0[tv]
                            w1s = w1[tv]
                            for j in range(8):
                                sl = pl.ds(j * L, L)
                                e = ent_rows[t, sl]
                                r = rel_rows[t, sl]
                                acc[sl] += w0s * e
                                acc[pl.ds(128 + j * L, L)] += w0s * r
                                acc[pl.ds(256 + j * L, L)] += w1s * e
                                acc[pl.ds(384 + j * L, L)] += w1s * r
                            return c2

                        lax.fori_loop(0, L, lane_body, 0)

                    return c

                lax.fori_loop(0, GPB2, dg_body, 0)

            return carry0

        lax.fori_loop(0, nsb, sb_body, 0)
        pltpu.sync_copy(acc, part_hbm.at[pl.ds(wid * 512, 512)])
        pltpu.sync_copy(cntv, cnt_hbm.at[pl.ds(wid * L, L)])

    return pl.kernel(
        body,
        out_type=[jax.ShapeDtypeStruct((NW * 512,), jnp.float32),
                  jax.ShapeDtypeStruct((NW * L,), jnp.float32)],
        mesh=mesh,
        compiler_params=pltpu.CompilerParams(needs_layout_passes=False),
        scratch_types=[
            pltpu.VMEM((2, epw), jnp.int32),
            pltpu.VMEM((800,), jnp.float32),
            pltpu.VMEM((L,), jnp.int32),
            pltpu.VMEM((L,), jnp.int32),
            pltpu.VMEM((256,), jnp.int32),
            pltpu.VMEM((L,), jnp.int32),
            pltpu.VMEM((L,), jnp.int32),
            pltpu.VMEM((L,), jnp.int32),
            pltpu.VMEM((L,), jnp.int32),
            pltpu.VMEM((L, D), jnp.float32),
            pltpu.VMEM((L, D), jnp.float32),
            pltpu.VMEM((512,), jnp.float32),
            pltpu.VMEM((L,), jnp.float32),
            pltpu.SemaphoreType.DMA,
            pltpu.SemaphoreType.DMA,
            pltpu.SemaphoreType.DMA,
            pltpu.SemaphoreType.DMA,
            pltpu.SemaphoreType.DMA,
            pltpu.SemaphoreType.DMA,
        ],
    )


@functools.lru_cache(maxsize=None)
def _build_sc(e_pad: int, nsb: int, sym: bool):
    epw = nsb * GPB * L
    mesh = plsc.VectorSubcoreMesh(
        core_axis_name="c", subcore_axis_name="s",
        num_cores=NC, num_subcores=NS)

    def body(src_hbm, typ_hbm, rel_hbm, uns_hbm, att_hbm,
             nid_hbm, ent_hbm, rtab_hbm,
             part_hbm, cnt_hbm,
             dstb, attv, unsv, idxbuf, srcbuf, typbuf, relbuf, nid16,
             ent_rows, rel_rows, acc, cntv,
             sem_s, sem_t, sem_r, sem_n, sem_e, sem_rr):
        wid = lax.axis_index("s") * NC + lax.axis_index("c")
        pltpu.sync_copy(att_hbm, attv)
        pltpu.sync_copy(uns_hbm, unsv)
        if sym:
            half = e_pad // 2
            shift = jnp.where(wid < NW // 2, half, -half)
        else:
            shift = e_pad
        pltpu.sync_copy(src_hbm.at[pl.ds(wid * epw + shift, epw)], dstb)
        zf = jnp.zeros((L,), jnp.float32)
        for j in range(32):
            acc[pl.ds(j * L, L)] = zf
        cntv[:] = zf
        unseen = unsv[:]
        lanes = lax.iota(jnp.int32, L)

        def sb_body(sb, carry0):
            base_g = sb * GPB

            def fast_body(k, m):
                base = (base_g + k * GU) * L
                for u in range(GU):
                    d = dstb[pl.ds(base + u * L, L)]
                    m = m | (d == unseen).astype(jnp.int32)
                return m

            orv = lax.fori_loop(0, GPB // GU, fast_body,
                                jnp.zeros((L,), jnp.int32))

            @pl.when(jnp.max(orv) > 0)
            def _rescan():
                def dg_body(g, c):
                    row = base_g + g
                    d = dstb[pl.ds(row * L, L)]
                    mask = d == unseen

                    @pl.when(jnp.any(mask))
                    def _hit_group():
                        ebase = wid * epw + row * L
                        idxbuf[:] = ebase + lanes
                        cp_s = pltpu.async_copy(src_hbm.at[idxbuf], srcbuf,
                                                sem_s)
                        cp_t = pltpu.async_copy(typ_hbm.at[idxbuf], typbuf,
                                                sem_t)
                        cp_r = pltpu.async_copy(rel_hbm.at[idxbuf], relbuf,
                                                sem_r)
                        cp_s.wait()
                        cp_t.wait()
                        cp_r.wait()
                        cp_n = pltpu.async_copy(nid_hbm.at[srcbuf], nid16,
                                                sem_n)
                        cp_rr = pltpu.async_copy(rtab_hbm.at[relbuf], rel_rows,
                                                 sem_rr)
                        cp_n.wait()
                        cp_e = pltpu.async_copy(ent_hbm.at[nid16], ent_rows,
                                                sem_e)
                        typv = typbuf[:]
                        maskf = jnp.where(mask, 1.0, 0.0)
                        w0 = plsc.load_gather(attv.at[:], [typv * 2]) * maskf
                        w1 = plsc.load_gather(attv.at[:], [typv * 2 + 1]) * maskf
                        cntv[:] = cntv[:] + maskf
                        cp_rr.wait()
                        cp_e.wait()

                        def lane_body(t, c2):
                            tv = jnp.full((L,), t, jnp.int32)
                            w0s = w0[tv]
                            w1s = w1[tv]
                            for j in range(8):
                                sl = pl.ds(j * L, L)
                                e = ent_rows[t, sl]
                                r = rel_rows[t, sl]
                                acc[sl] += w0s * e
                                acc[pl.ds(128 + j * L, L)] += w0s * r
                                acc[pl.ds(256 + j * L, L)] += w1s * e
                                acc[pl.ds(384 + j * L, L)] += w1s * r
                            return c2

                        lax.fori_loop(0, L, lane_body, 0)

                    return c

                lax.fori_loop(0, GPB, dg_body, 0)

            return carry0

        lax.fori_loop(0, nsb, sb_body, 0)
        pltpu.sync_copy(acc, part_hbm.at[pl.ds(wid * 512, 512)])
        pltpu.sync_copy(cntv, cnt_hbm.at[pl.ds(wid * L, L)])

    return pl.kernel(
        body,
        out_type=[jax.ShapeDtypeStruct((NW * 512,), jnp.float32),
                  jax.ShapeDtypeStruct((NW * L,), jnp.float32)],
        mesh=mesh,
        compiler_params=pltpu.CompilerParams(needs_layout_passes=False),
        scratch_types=[
            pltpu.VMEM((epw,), jnp.int32),
            pltpu.VMEM((800,), jnp.float32),
            pltpu.VMEM((L,), jnp.int32),
            pltpu.VMEM((L,), jnp.int32),
            pltpu.VMEM((L,), jnp.int32),
            pltpu.VMEM((L,), jnp.int32),
            pltpu.VMEM((L,), jnp.int32),
            pltpu.VMEM((L,), jnp.int32),
            pltpu.VMEM((L, D), jnp.float32),
            pltpu.VMEM((L, D), jnp.float32),
            pltpu.VMEM((512,), jnp.float32),
            pltpu.VMEM((L,), jnp.float32),
            pltpu.SemaphoreType.DMA,
            pltpu.SemaphoreType.DMA,
            pltpu.SemaphoreType.DMA,
            pltpu.SemaphoreType.DMA,
            pltpu.SemaphoreType.DMA,
            pltpu.SemaphoreType.DMA,
        ],
    )


def _combine_body(p_ref, c_ref, b_ref, o_ref):
    p = p_ref[...]
    v4 = jnp.sum(p.reshape(NW, 4, D), axis=0)
    cnt = jnp.sum(c_ref[...])
    v0 = jnp.concatenate([v4[0:1], v4[1:2]], axis=1)
    v1 = jnp.concatenate([v4[2:3], v4[3:4]], axis=1)
    b = b_ref[...]
    r = jnp.dot(v0, b[0], preferred_element_type=jnp.float32)
    r = r + jnp.dot(v1, b[1], preferred_element_type=jnp.float32)
    o_ref[...] = r / jnp.maximum(cnt, 1.0)


_combine = pl.pallas_call(
    _combine_body,
    out_shape=jax.ShapeDtypeStruct((1, D), jnp.float32),
)


def kernel(node_id, edge_index, edge_type, rel_index, unseen_index,
           entity_embedding, relation_embedding, att, basis):
    e = edge_type.shape[0]
    uns0 = jnp.full((L,), unseen_index, dtype=jnp.int32)
    att0 = att.reshape(-1)
    if e % 12800 == 0:
        nsb2 = -(-e // (NW * GPB2 * L))
        part, cnt = _build_sc_raw(e, nsb2)(
            edge_index, edge_type, rel_index, uns0, att0,
            node_id, entity_embedding, relation_embedding)
        out = _combine(part.reshape(D, D), cnt.reshape(4, D), basis)
        return out[0]
    spb = GPB * L
    nsb = -(-e // (NW * spb))
    e_pad = NW * nsb * spb
    pe = e_pad - e
    if pe == 0:
        src = edge_index.reshape(-1)
        typ = edge_type
        rel = rel_index
        sym = False
    else:
        srcp = jnp.pad(edge_index[0], (0, pe))
        dstp = jnp.pad(edge_index[1], (0, pe), constant_values=-1)
        src = jnp.concatenate([srcp, dstp])
        typ = jnp.pad(edge_type, (0, pe))
        rel = jnp.pad(rel_index, (0, pe))
        sym = False
    part, cnt = _build_sc(e_pad, nsb, sym)(
        src, typ, rel, uns0, att0,
        node_id, entity_embedding, relation_embedding)
    out = _combine(part.reshape(D, D), cnt.reshape(4, D), basis)
    return out[0]
